# baseline (speedup 1.0000x reference)
import functools

import jax
import jax.numpy as jnp
from jax.experimental import pallas as pl

_BETA = 0.25
_VQK = 512
_VQD = 64


def _mm_kernel(x_ref, w_ref, b_ref, o_ref, *, relu):
    acc = jnp.dot(x_ref[...], w_ref[...], preferred_element_type=jnp.float32)
    acc = acc + b_ref[...]
    if relu:
        acc = jnp.maximum(acc, 0.0)
    o_ref[...] = acc


def _mm(x, w, b, relu, bm=512):
    m, k = x.shape
    n = w.shape[1]
    return pl.pallas_call(
        functools.partial(_mm_kernel, relu=relu),
        grid=(pl.cdiv(m, bm),),
        in_specs=[
            pl.BlockSpec((bm, k), lambda i: (i, 0)),
            pl.BlockSpec((k, n), lambda i: (0, 0)),
            pl.BlockSpec((1, n), lambda i: (0, 0)),
        ],
        out_specs=pl.BlockSpec((bm, n), lambda i: (i, 0)),
        out_shape=jax.ShapeDtypeStruct((m, n), jnp.float32),
    )(x, w, b.reshape(1, n))


def _conv3_kernel(x0_ref, x1_ref, x2_ref, w_ref, b_ref, o_ref, *, wo, relu):
    rows = (x0_ref[0, 0], x1_ref[0, 0], x2_ref[0, 0])
    acc = None
    for a in range(3):
        row = rows[a]
        for c in range(3):
            xm = jax.lax.slice_in_dim(row, c, c + wo, axis=0)
            t = jnp.dot(xm, w_ref[3 * a + c], preferred_element_type=jnp.float32)
            acc = t if acc is None else acc + t
    acc = acc + b_ref[...]
    if relu:
        acc = jnp.maximum(acc, 0.0)
    o_ref[0, 0] = acc


def _conv3(xpad, w9, b, relu):
    bsz, hp, wp, cin = xpad.shape
    ho, wo = hp - 2, wp - 2
    cout = w9.shape[2]
    return pl.pallas_call(
        functools.partial(_conv3_kernel, wo=wo, relu=relu),
        grid=(bsz, ho),
        in_specs=[
            pl.BlockSpec((1, 1, wp, cin), lambda i, r: (i, r, 0, 0)),
            pl.BlockSpec((1, 1, wp, cin), lambda i, r: (i, r + 1, 0, 0)),
            pl.BlockSpec((1, 1, wp, cin), lambda i, r: (i, r + 2, 0, 0)),
            pl.BlockSpec((9, cin, cout), lambda i, r: (0, 0, 0)),
            pl.BlockSpec((1, cout), lambda i, r: (0, 0)),
        ],
        out_specs=pl.BlockSpec((1, 1, wo, cout), lambda i, r: (i, r, 0, 0)),
        out_shape=jax.ShapeDtypeStruct((bsz, ho, wo, cout), jnp.float32),
    )(xpad, xpad, xpad, w9, b.reshape(1, cout))


def _vq_kernel(z_ref, cbt_ref, cb_ref, idx_ref, zq_ref, cnt_ref, loss_ref,
               perp_ref, *, nblocks, m_total):
    i = pl.program_id(0)
    z = z_ref[...]
    cbt = cbt_ref[...]
    zn = jnp.sum(z * z, axis=1, keepdims=True)
    cn = jnp.sum(cbt * cbt, axis=0, keepdims=True)
    mm = jnp.dot(z, cbt, preferred_element_type=jnp.float32)
    d = zn + cn - 2.0 * mm
    dmin = jnp.min(d, axis=1, keepdims=True)
    col = jax.lax.broadcasted_iota(jnp.int32, d.shape, 1)
    idx = jnp.min(jnp.where(d == dmin, col, _VQK), axis=1, keepdims=True)
    idx_ref[...] = idx
    onehot = (col == idx).astype(jnp.float32)
    zq = jnp.dot(onehot, cb_ref[...], preferred_element_type=jnp.float32)
    zq_ref[...] = zq
    diff = zq - z
    sq = jnp.sum(jnp.sum(diff * diff, axis=1, keepdims=True),
                 axis=0, keepdims=True)
    cnt = jnp.sum(onehot, axis=0, keepdims=True)

    @pl.when(i == 0)
    def _init():
        cnt_ref[...] = cnt
        loss_ref[...] = sq

    @pl.when(i > 0)
    def _accum():
        cnt_ref[...] = cnt_ref[...] + cnt
        loss_ref[...] = loss_ref[...] + sq

    @pl.when(i == nblocks - 1)
    def _finalize():
        loss_ref[...] = (1.0 + _BETA) * loss_ref[...] / (m_total * _VQD)
        e = cnt_ref[...] / m_total
        ent = jnp.sum(e * jnp.log(e + 1e-10), axis=1, keepdims=True)
        perp_ref[...] = jnp.exp(-ent)


def _vq(zflat, codebook, bm=896):
    m = zflat.shape[0]
    nblocks = m // bm
    assert nblocks * bm == m
    kern = functools.partial(_vq_kernel, nblocks=nblocks, m_total=m)
    idx, zq, _cnt, loss, perp = pl.pallas_call(
        kern,
        grid=(nblocks,),
        in_specs=[
            pl.BlockSpec((bm, _VQD), lambda i: (i, 0)),
            pl.BlockSpec((_VQD, _VQK), lambda i: (0, 0)),
            pl.BlockSpec((_VQK, _VQD), lambda i: (0, 0)),
        ],
        out_specs=[
            pl.BlockSpec((bm, 1), lambda i: (i, 0)),
            pl.BlockSpec((bm, _VQD), lambda i: (i, 0)),
            pl.BlockSpec((1, _VQK), lambda i: (0, 0)),
            pl.BlockSpec((1, 1), lambda i: (0, 0)),
            pl.BlockSpec((1, 1), lambda i: (0, 0)),
        ],
        out_shape=[
            jax.ShapeDtypeStruct((m, 1), jnp.int32),
            jax.ShapeDtypeStruct((m, _VQD), jnp.float32),
            jax.ShapeDtypeStruct((1, _VQK), jnp.float32),
            jax.ShapeDtypeStruct((1, 1), jnp.float32),
            jax.ShapeDtypeStruct((1, 1), jnp.float32),
        ],
    )(zflat, codebook.T, codebook)
    return idx, zq, loss, perp


def _w_conv(w):
    o, i, _, _ = w.shape
    return jnp.transpose(w, (2, 3, 1, 0)).reshape(9, i, o)


def _w_im2col(w):
    o = w.shape[0]
    return jnp.transpose(w, (2, 3, 1, 0)).reshape(-1, o)


def kernel(x, enc_w1, enc_b1, enc_w2, enc_b2, enc_w3, enc_b3, enc_w4, enc_b4,
           codebook, dec_w1, dec_b1, dec_w2, dec_b2, dec_w3, dec_b3, dec_w4,
           dec_b4):
    bsz = x.shape[0]
    img = x.shape[2]
    h1s = img // 2
    h2s = h1s // 2

    xp = jnp.pad(x[:, 0], ((0, 0), (1, 1), (1, 1)))
    pat = jnp.stack(
        [xp[:, a:a + 2 * h1s:2, c:c + 2 * h1s:2] for a in range(3) for c in range(3)],
        axis=-1).reshape(bsz * h1s * h1s, 9)
    h1 = _mm(pat, _w_im2col(enc_w1), enc_b1, relu=True)
    h1 = h1.reshape(bsz, h1s, h1s, 128)

    h1p = jnp.pad(h1, ((0, 0), (1, 1), (1, 1), (0, 0)))
    pat2 = jnp.stack(
        [h1p[:, a:a + 2 * h2s:2, c:c + 2 * h2s:2, :] for a in range(3) for c in range(3)],
        axis=3).reshape(bsz * h2s * h2s, 9 * 128)
    h2 = _mm(pat2, _w_im2col(enc_w2), enc_b2, relu=True)
    h2 = h2.reshape(bsz, h2s, h2s, 128)

    h3 = _conv3(jnp.pad(h2, ((0, 0), (1, 1), (1, 1), (0, 0))),
                _w_conv(enc_w3), enc_b3, relu=True)
    z = _conv3(jnp.pad(h3, ((0, 0), (1, 1), (1, 1), (0, 0))),
               _w_conv(enc_w4), enc_b4, relu=True)

    zflat = z.reshape(bsz * h2s * h2s, _VQD)
    idx, zq, loss, perp = _vq(zflat, codebook)
    idxs = idx.reshape(bsz, h2s, h2s)

    q = h2s + 1
    zq4 = zq.reshape(bsz, h2s, h2s, _VQD)
    zqp = jnp.pad(zq4, ((0, 0), (1, 1), (1, 1), (0, 0)))
    taps = jnp.stack(
        [zqp[:, 1 - a:1 - a + q, 1 - c:1 - c + q, :] for a in range(2) for c in range(2)],
        axis=3).reshape(bsz * q * q, 4 * _VQD)
    wf = dec_w1[:, :, ::-1, ::-1]
    t6 = jnp.transpose(wf, (2, 3, 1, 0)).reshape(2, 3, 2, 3, _VQD, 128)
    wd1 = jnp.transpose(t6, (0, 2, 4, 1, 3, 5)).reshape(4 * _VQD, 9 * 128)
    bd1 = jnp.tile(dec_b1, 9)
    y1 = _mm(taps, wd1, bd1, relu=True)
    y1 = y1.reshape(bsz, q, q, 3, 3, 128)
    y1 = jnp.transpose(y1, (0, 1, 3, 2, 4, 5)).reshape(bsz, 3 * q, 3 * q, 128)

    y2 = _conv3(jnp.pad(y1, ((0, 0), (2, 2), (2, 2), (0, 0))),
                _w_conv(dec_w2), dec_b2, relu=True)
    y3 = _conv3(jnp.pad(y2, ((0, 0), (2, 2), (2, 2), (0, 0))),
                _w_conv(dec_w3), dec_b3, relu=True)

    hw = y3.shape[1]
    w4 = jnp.zeros((128, 128), jnp.float32).at[:, 0].set(dec_w4[0, :, 0, 0])
    b4 = jnp.zeros((128,), jnp.float32).at[0].set(dec_b4[0])
    y4 = _mm(y3.reshape(bsz * hw * hw, 128), w4, b4, relu=False)
    decoded = y4[:, :1].reshape(bsz, hw, hw, 1).transpose(0, 3, 1, 2)

    return loss.reshape(()), decoded, perp.reshape(()), idxs


# speedup vs baseline: 1.3378x; 1.3378x over previous
import functools

import jax
import jax.numpy as jnp
from jax.experimental import pallas as pl

_BETA = 0.25
_VQK = 512
_VQD = 64


def _mm_kernel(x_ref, w_ref, b_ref, o_ref, *, relu, out_dtype):
    acc = jnp.dot(x_ref[...], w_ref[...], preferred_element_type=jnp.float32)
    acc = acc + b_ref[...]
    if relu:
        acc = jnp.maximum(acc, 0.0)
    o_ref[...] = acc.astype(out_dtype)


def _mm(x, w, b, relu, bm=512, out_dtype=jnp.float32):
    m, k = x.shape
    n = w.shape[1]
    return pl.pallas_call(
        functools.partial(_mm_kernel, relu=relu, out_dtype=out_dtype),
        grid=(pl.cdiv(m, bm),),
        in_specs=[
            pl.BlockSpec((bm, k), lambda i: (i, 0)),
            pl.BlockSpec((k, n), lambda i: (0, 0)),
            pl.BlockSpec((1, n), lambda i: (0, 0)),
        ],
        out_specs=pl.BlockSpec((bm, n), lambda i: (i, 0)),
        out_shape=jax.ShapeDtypeStruct((m, n), out_dtype),
    )(x, w, b.reshape(1, n))


def _conv3f_kernel(xg_ref, w_ref, b_ref, o_ref, *, wp, m, relu, out_dtype):
    slab = xg_ref[0, 0]
    acc = None
    for a in range(3):
        for c in range(3):
            xm = jax.lax.slice_in_dim(slab, a * wp + c, a * wp + c + m, axis=0)
            t = jnp.dot(xm, w_ref[3 * a + c], preferred_element_type=jnp.float32)
            acc = t if acc is None else acc + t
    acc = acc + b_ref[...]
    if relu:
        acc = jnp.maximum(acc, 0.0)
    o_ref[0, 0, :m, :] = acc.astype(out_dtype)


def _conv3(xpad, w9, b, relu, bh, out_dtype=jnp.float32):
    bsz, hp, wp0, cin = xpad.shape
    ho, wo = hp - 2, wp0 - 2
    cout = w9.shape[2]
    wp = -(-wp0 // 8) * 8
    nh = -(-ho // bh)
    hg = nh * bh
    xp2 = jnp.pad(xpad, ((0, 0), (0, hg + 2 - hp), (0, wp - wp0), (0, 0)))
    flat = xp2.reshape(bsz, (hg + 2) * wp, cin)
    xg = jnp.stack(
        [flat[:, i * bh * wp:(i * bh + bh + 2) * wp] for i in range(nh)], axis=1)
    m = bh * wp - 2
    slab = (bh + 2) * wp
    out = pl.pallas_call(
        functools.partial(_conv3f_kernel, wp=wp, m=m, relu=relu,
                          out_dtype=out_dtype),
        grid=(bsz, nh),
        in_specs=[
            pl.BlockSpec((1, 1, slab, cin), lambda i, j: (i, j, 0, 0)),
            pl.BlockSpec((9, cin, cout), lambda i, j: (0, 0, 0)),
            pl.BlockSpec((1, cout), lambda i, j: (0, 0)),
        ],
        out_specs=pl.BlockSpec((1, 1, bh * wp, cout), lambda i, j: (i, j, 0, 0)),
        out_shape=jax.ShapeDtypeStruct((bsz, nh, bh * wp, cout), out_dtype),
    )(xg, w9, b.reshape(1, cout))
    return out.reshape(bsz, hg, wp, cout)[:, :ho, :wo, :]


def _vq_kernel(z_ref, cbt_ref, cb_ref, idx_ref, zq_ref, cnt_ref, loss_ref,
               perp_ref, *, nblocks, m_total):
    i = pl.program_id(0)
    z = z_ref[...]
    cbt = cbt_ref[...]
    zn = jnp.sum(z * z, axis=1, keepdims=True)
    cn = jnp.sum(cbt * cbt, axis=0, keepdims=True)
    mm = jnp.dot(z, cbt, preferred_element_type=jnp.float32)
    d = zn + cn - 2.0 * mm
    dmin = jnp.min(d, axis=1, keepdims=True)
    col = jax.lax.broadcasted_iota(jnp.int32, d.shape, 1)
    idx = jnp.min(jnp.where(d == dmin, col, _VQK), axis=1, keepdims=True)
    idx_ref[...] = idx
    onehot = (col == idx).astype(jnp.float32)
    zq = jnp.dot(onehot, cb_ref[...], preferred_element_type=jnp.float32)
    zq_ref[...] = zq
    diff = zq - z
    sq = jnp.sum(jnp.sum(diff * diff, axis=1, keepdims=True),
                 axis=0, keepdims=True)
    cnt = jnp.sum(onehot, axis=0, keepdims=True)

    @pl.when(i == 0)
    def _init():
        cnt_ref[...] = cnt
        loss_ref[...] = sq

    @pl.when(i > 0)
    def _accum():
        cnt_ref[...] = cnt_ref[...] + cnt
        loss_ref[...] = loss_ref[...] + sq

    @pl.when(i == nblocks - 1)
    def _finalize():
        loss_ref[...] = (1.0 + _BETA) * loss_ref[...] / (m_total * _VQD)
        e = cnt_ref[...] / m_total
        ent = jnp.sum(e * jnp.log(e + 1e-10), axis=1, keepdims=True)
        perp_ref[...] = jnp.exp(-ent)


def _vq(zflat, codebook, bm=896):
    m = zflat.shape[0]
    nblocks = m // bm
    assert nblocks * bm == m
    kern = functools.partial(_vq_kernel, nblocks=nblocks, m_total=m)
    idx, zq, _cnt, loss, perp = pl.pallas_call(
        kern,
        grid=(nblocks,),
        in_specs=[
            pl.BlockSpec((bm, _VQD), lambda i: (i, 0)),
            pl.BlockSpec((_VQD, _VQK), lambda i: (0, 0)),
            pl.BlockSpec((_VQK, _VQD), lambda i: (0, 0)),
        ],
        out_specs=[
            pl.BlockSpec((bm, 1), lambda i: (i, 0)),
            pl.BlockSpec((bm, _VQD), lambda i: (i, 0)),
            pl.BlockSpec((1, _VQK), lambda i: (0, 0)),
            pl.BlockSpec((1, 1), lambda i: (0, 0)),
            pl.BlockSpec((1, 1), lambda i: (0, 0)),
        ],
        out_shape=[
            jax.ShapeDtypeStruct((m, 1), jnp.int32),
            jax.ShapeDtypeStruct((m, _VQD), jnp.float32),
            jax.ShapeDtypeStruct((1, _VQK), jnp.float32),
            jax.ShapeDtypeStruct((1, 1), jnp.float32),
            jax.ShapeDtypeStruct((1, 1), jnp.float32),
        ],
    )(zflat, codebook.T, codebook)
    return idx, zq, loss, perp


def _w_conv(w, dtype=jnp.float32):
    o, i, _, _ = w.shape
    return jnp.transpose(w, (2, 3, 1, 0)).reshape(9, i, o).astype(dtype)


def _w_im2col(w):
    o = w.shape[0]
    return jnp.transpose(w, (2, 3, 1, 0)).reshape(-1, o)


def kernel(x, enc_w1, enc_b1, enc_w2, enc_b2, enc_w3, enc_b3, enc_w4, enc_b4,
           codebook, dec_w1, dec_b1, dec_w2, dec_b2, dec_w3, dec_b3, dec_w4,
           dec_b4):
    bsz = x.shape[0]
    img = x.shape[2]
    h1s = img // 2
    h2s = h1s // 2
    bf16 = jnp.bfloat16

    xp = jnp.pad(x[:, 0], ((0, 0), (1, 1), (1, 1)))
    pat = jnp.stack(
        [xp[:, a:a + 2 * h1s:2, c:c + 2 * h1s:2] for a in range(3) for c in range(3)],
        axis=-1).reshape(bsz * h1s * h1s, 9)
    h1 = _mm(pat, _w_im2col(enc_w1), enc_b1, relu=True)
    h1 = h1.reshape(bsz, h1s, h1s, 128)

    h1p = jnp.pad(h1, ((0, 0), (1, 1), (1, 1), (0, 0)))
    pat2 = jnp.stack(
        [h1p[:, a:a + 2 * h2s:2, c:c + 2 * h2s:2, :] for a in range(3) for c in range(3)],
        axis=3).reshape(bsz * h2s * h2s, 9 * 128)
    h2 = _mm(pat2, _w_im2col(enc_w2), enc_b2, relu=True)
    h2 = h2.reshape(bsz, h2s, h2s, 128)

    h3 = _conv3(jnp.pad(h2, ((0, 0), (1, 1), (1, 1), (0, 0))),
                _w_conv(enc_w3), enc_b3, relu=True, bh=28)
    z = _conv3(jnp.pad(h3, ((0, 0), (1, 1), (1, 1), (0, 0))),
               _w_conv(enc_w4), enc_b4, relu=True, bh=28)

    zflat = z.reshape(bsz * h2s * h2s, _VQD)
    idx, zq, loss, perp = _vq(zflat, codebook)
    idxs = idx.reshape(bsz, h2s, h2s)

    q = h2s + 1
    zq4 = zq.reshape(bsz, h2s, h2s, _VQD)
    zqp = jnp.pad(zq4, ((0, 0), (1, 1), (1, 1), (0, 0))).astype(bf16)
    taps = jnp.stack(
        [zqp[:, 1 - a:1 - a + q, 1 - c:1 - c + q, :] for a in range(2) for c in range(2)],
        axis=3).reshape(bsz * q * q, 4 * _VQD)
    wf = dec_w1[:, :, ::-1, ::-1]
    t6 = jnp.transpose(wf, (2, 3, 1, 0)).reshape(2, 3, 2, 3, _VQD, 128)
    wd1 = jnp.transpose(t6, (0, 2, 4, 1, 3, 5)).reshape(4 * _VQD, 9 * 128)
    bd1 = jnp.tile(dec_b1, 9)
    y1 = _mm(taps, wd1.astype(bf16), bd1, relu=True, out_dtype=bf16)
    y1 = y1.reshape(bsz, q, q, 3, 3, 128)
    y1 = jnp.transpose(y1, (0, 1, 3, 2, 4, 5)).reshape(bsz, 3 * q, 3 * q, 128)

    y2 = _conv3(jnp.pad(y1, ((0, 0), (2, 2), (2, 2), (0, 0))),
                _w_conv(dec_w2, bf16), dec_b2, relu=True, bh=32, out_dtype=bf16)
    y3 = _conv3(jnp.pad(y2, ((0, 0), (2, 2), (2, 2), (0, 0))),
                _w_conv(dec_w3, bf16), dec_b3, relu=True, bh=32, out_dtype=bf16)

    hw = y3.shape[1]
    w4 = jnp.zeros((128, 128), jnp.float32).at[:, 0].set(dec_w4[0, :, 0, 0])
    b4 = jnp.zeros((128,), jnp.float32).at[0].set(dec_b4[0])
    y4 = _mm(y3.reshape(bsz * hw * hw, 128), w4.astype(bf16), b4, relu=False)
    decoded = y4[:, :1].reshape(bsz, hw, hw, 1).transpose(0, 3, 1, 2)

    return loss.reshape(()), decoded, perp.reshape(()), idxs


# speedup vs baseline: 2.5362x; 1.8958x over previous
import functools

import jax
import jax.numpy as jnp
from jax.experimental import pallas as pl
from jax.experimental.pallas import tpu as pltpu

_BETA = 0.25
_VQK = 512
_VQD = 64


def _mm_kernel(x_ref, w_ref, b_ref, o_ref, *, relu, out_dtype):
    acc = jnp.dot(x_ref[...], w_ref[...], preferred_element_type=jnp.float32)
    acc = acc + b_ref[...]
    if relu:
        acc = jnp.maximum(acc, 0.0)
    o_ref[...] = acc.astype(out_dtype)


def _mm(x, w, b, relu, bm=512, out_dtype=jnp.float32):
    m, k = x.shape
    n = w.shape[1]
    return pl.pallas_call(
        functools.partial(_mm_kernel, relu=relu, out_dtype=out_dtype),
        grid=(pl.cdiv(m, bm),),
        in_specs=[
            pl.BlockSpec((bm, k), lambda i: (i, 0)),
            pl.BlockSpec((k, n), lambda i: (0, 0)),
            pl.BlockSpec((1, n), lambda i: (0, 0)),
        ],
        out_specs=pl.BlockSpec((bm, n), lambda i: (i, 0)),
        out_shape=jax.ShapeDtypeStruct((m, n), out_dtype),
    )(x, w, b.reshape(1, n))


def _enc2_kernel(x0_ref, x1_ref, x2_ref, s_ref, w_ref, b_ref, o_ref):
    acc = None
    for a, r in enumerate((x0_ref, x1_ref, x2_ref)):
        u = r[0, 0]
        for c in range(3):
            t = jnp.dot(s_ref[c], u, preferred_element_type=jnp.float32)
            t = jnp.dot(t, w_ref[3 * a + c], preferred_element_type=jnp.float32)
            acc = t if acc is None else acc + t
    o_ref[0, 0] = jnp.maximum(acc + b_ref[...], 0.0)


def _conv_s2(h1p, w9, b, ho):
    bsz, _, wp_in, cin = h1p.shape
    cout = w9.shape[2]
    col = jax.lax.broadcasted_iota(jnp.int32, (3, ho, wp_in), 2)
    tgt = 2 * jax.lax.broadcasted_iota(jnp.int32, (3, ho, wp_in), 1) + \
        jax.lax.broadcasted_iota(jnp.int32, (3, ho, wp_in), 0)
    sel = (col == tgt).astype(jnp.float32)
    return pl.pallas_call(
        _enc2_kernel,
        grid=(bsz, ho),
        in_specs=[
            pl.BlockSpec((1, 1, wp_in, cin), lambda i, r: (i, 2 * r, 0, 0)),
            pl.BlockSpec((1, 1, wp_in, cin), lambda i, r: (i, 2 * r + 1, 0, 0)),
            pl.BlockSpec((1, 1, wp_in, cin), lambda i, r: (i, 2 * r + 2, 0, 0)),
            pl.BlockSpec((3, ho, wp_in), lambda i, r: (0, 0, 0)),
            pl.BlockSpec((9, cin, cout), lambda i, r: (0, 0, 0)),
            pl.BlockSpec((1, cout), lambda i, r: (0, 0)),
        ],
        out_specs=pl.BlockSpec((1, 1, ho, cout), lambda i, r: (i, r, 0, 0)),
        out_shape=jax.ShapeDtypeStruct((bsz, ho, ho, cout), jnp.float32),
    )(h1p, h1p, h1p, sel, w9, b.reshape(1, cout))


def _conv3f_kernel(x_ref, w_ref, b_ref, o_ref, scr, sem, *, wp, bh, m, slab,
                   relu, out_dtype):
    bi = pl.program_id(0)
    j = pl.program_id(1)
    cp = pltpu.make_async_copy(x_ref.at[bi, pl.ds(j * bh * wp, slab)], scr, sem)
    cp.start()
    cp.wait()
    acc = None
    for a in range(3):
        for c in range(3):
            off = a * wp + c
            xm = scr[off:off + m, :]
            t = jnp.dot(xm, w_ref[3 * a + c], preferred_element_type=jnp.float32)
            acc = t if acc is None else acc + t
    acc = acc + b_ref[...]
    if relu:
        acc = jnp.maximum(acc, 0.0)
    o_ref[0, 0, :m, :] = acc.astype(out_dtype)


def _conv3(xpad, w9, b, relu, bh, out_dtype=jnp.float32):
    bsz, hp, wp0, cin = xpad.shape
    ho = hp - 2
    cout = w9.shape[2]
    wp = -(-wp0 // 8) * 8
    nh = -(-ho // bh)
    hg = nh * bh
    xp2 = jnp.pad(xpad, ((0, 0), (0, hg + 2 - hp), (0, wp - wp0), (0, 0)))
    flat = xp2.reshape(bsz, (hg + 2) * wp, cin)
    m = bh * wp - 2
    slab = (bh + 2) * wp
    out = pl.pallas_call(
        functools.partial(_conv3f_kernel, wp=wp, bh=bh, m=m, slab=slab,
                          relu=relu, out_dtype=out_dtype),
        grid=(bsz, nh),
        in_specs=[
            pl.BlockSpec(memory_space=pl.ANY),
            pl.BlockSpec((9, cin, cout), lambda i, j: (0, 0, 0)),
            pl.BlockSpec((1, cout), lambda i, j: (0, 0)),
        ],
        out_specs=pl.BlockSpec((1, 1, bh * wp, cout), lambda i, j: (i, j, 0, 0)),
        out_shape=jax.ShapeDtypeStruct((bsz, nh, bh * wp, cout), out_dtype),
        scratch_shapes=[pltpu.VMEM((slab, cin), xpad.dtype),
                        pltpu.SemaphoreType.DMA],
    )(flat, w9, b.reshape(1, cout))
    return out.reshape(bsz, hg, wp, cout)


def _dec1_kernel(x_ref, w_ref, b_ref, o_ref, scr, sem, *, wp, bh, m, slab):
    bi = pl.program_id(0)
    j = pl.program_id(1)
    cp = pltpu.make_async_copy(x_ref.at[bi, pl.ds(j * bh * wp, slab)], scr, sem)
    cp.start()
    cp.wait()
    offs = (wp + 1, wp, 1, 0)
    acc = None
    for t in range(4):
        xm = scr[offs[t]:offs[t] + m, :]
        u = jnp.dot(xm, w_ref[t], preferred_element_type=jnp.float32)
        acc = u if acc is None else acc + u
    acc = jnp.maximum(acc + b_ref[...], 0.0)
    o_ref[0, 0, :m, :] = acc.astype(jnp.bfloat16)


def _dec1(zqp_flat, wd1, bd1, bsz, wp, bh, nh):
    m = bh * wp - 1
    slab = (bh + 1) * wp
    return pl.pallas_call(
        functools.partial(_dec1_kernel, wp=wp, bh=bh, m=m, slab=slab),
        grid=(bsz, nh),
        in_specs=[
            pl.BlockSpec(memory_space=pl.ANY),
            pl.BlockSpec((4, _VQD, 1152), lambda i, j: (0, 0, 0)),
            pl.BlockSpec((1, 1152), lambda i, j: (0, 0)),
        ],
        out_specs=pl.BlockSpec((1, 1, bh * wp, 1152), lambda i, j: (i, j, 0, 0)),
        out_shape=jax.ShapeDtypeStruct((bsz, nh, bh * wp, 1152), jnp.bfloat16),
        scratch_shapes=[pltpu.VMEM((slab, _VQD), jnp.bfloat16),
                        pltpu.SemaphoreType.DMA],
    )(zqp_flat, wd1, bd1.reshape(1, 1152))


def _vq_kernel(z_ref, cbt_ref, cb_ref, idx_ref, zq_ref, cnt_ref, loss_ref,
               perp_ref, *, nblocks, m_total):
    i = pl.program_id(0)
    z = z_ref[...]
    cbt = cbt_ref[...]
    zn = jnp.sum(z * z, axis=1, keepdims=True)
    cn = jnp.sum(cbt * cbt, axis=0, keepdims=True)
    mm = jnp.dot(z, cbt, preferred_element_type=jnp.float32)
    d = zn + cn - 2.0 * mm
    dmin = jnp.min(d, axis=1, keepdims=True)
    col = jax.lax.broadcasted_iota(jnp.int32, d.shape, 1)
    idx = jnp.min(jnp.where(d == dmin, col, _VQK), axis=1, keepdims=True)
    idx_ref[...] = idx
    onehot = (col == idx).astype(jnp.float32)
    zq = jnp.dot(onehot, cb_ref[...], preferred_element_type=jnp.float32)
    zq_ref[...] = zq
    diff = zq - z
    sq = jnp.sum(jnp.sum(diff * diff, axis=1, keepdims=True),
                 axis=0, keepdims=True)
    cnt = jnp.sum(onehot, axis=0, keepdims=True)

    @pl.when(i == 0)
    def _init():
        cnt_ref[...] = cnt
        loss_ref[...] = sq

    @pl.when(i > 0)
    def _accum():
        cnt_ref[...] = cnt_ref[...] + cnt
        loss_ref[...] = loss_ref[...] + sq

    @pl.when(i == nblocks - 1)
    def _finalize():
        loss_ref[...] = (1.0 + _BETA) * loss_ref[...] / (m_total * _VQD)
        e = cnt_ref[...] / m_total
        ent = jnp.sum(e * jnp.log(e + 1e-10), axis=1, keepdims=True)
        perp_ref[...] = jnp.exp(-ent)


def _vq(zflat, codebook, bm=896):
    m = zflat.shape[0]
    nblocks = m // bm
    assert nblocks * bm == m
    kern = functools.partial(_vq_kernel, nblocks=nblocks, m_total=m)
    idx, zq, _cnt, loss, perp = pl.pallas_call(
        kern,
        grid=(nblocks,),
        in_specs=[
            pl.BlockSpec((bm, _VQD), lambda i: (i, 0)),
            pl.BlockSpec((_VQD, _VQK), lambda i: (0, 0)),
            pl.BlockSpec((_VQK, _VQD), lambda i: (0, 0)),
        ],
        out_specs=[
            pl.BlockSpec((bm, 1), lambda i: (i, 0)),
            pl.BlockSpec((bm, _VQD), lambda i: (i, 0)),
            pl.BlockSpec((1, _VQK), lambda i: (0, 0)),
            pl.BlockSpec((1, 1), lambda i: (0, 0)),
            pl.BlockSpec((1, 1), lambda i: (0, 0)),
        ],
        out_shape=[
            jax.ShapeDtypeStruct((m, 1), jnp.int32),
            jax.ShapeDtypeStruct((m, _VQD), jnp.float32),
            jax.ShapeDtypeStruct((1, _VQK), jnp.float32),
            jax.ShapeDtypeStruct((1, 1), jnp.float32),
            jax.ShapeDtypeStruct((1, 1), jnp.float32),
        ],
    )(zflat, codebook.T, codebook)
    return idx, zq, loss, perp


def _w_conv(w, dtype=jnp.float32):
    o, i, _, _ = w.shape
    return jnp.transpose(w, (2, 3, 1, 0)).reshape(9, i, o).astype(dtype)


def _w_im2col(w):
    o = w.shape[0]
    return jnp.transpose(w, (2, 3, 1, 0)).reshape(-1, o)


def kernel(x, enc_w1, enc_b1, enc_w2, enc_b2, enc_w3, enc_b3, enc_w4, enc_b4,
           codebook, dec_w1, dec_b1, dec_w2, dec_b2, dec_w3, dec_b3, dec_w4,
           dec_b4):
    bsz = x.shape[0]
    img = x.shape[2]
    h1s = img // 2
    h2s = h1s // 2
    bf16 = jnp.bfloat16

    xp = jnp.pad(x[:, 0], ((0, 0), (1, 1), (1, 1)))
    pat = jnp.stack(
        [xp[:, a:a + 2 * h1s:2, c:c + 2 * h1s:2] for a in range(3) for c in range(3)],
        axis=-1).reshape(bsz * h1s * h1s, 9)
    h1 = _mm(pat, _w_im2col(enc_w1), enc_b1, relu=True)
    h1 = h1.reshape(bsz, h1s, h1s, 128)

    h1p = jnp.pad(h1, ((0, 0), (1, 1), (1, 7), (0, 0)))
    h2 = _conv_s2(h1p, _w_conv(enc_w2), enc_b2, h2s)

    h3g = _conv3(jnp.pad(h2, ((0, 0), (1, 1), (1, 1), (0, 0))),
                 _w_conv(enc_w3), enc_b3, relu=True, bh=28)
    zg = _conv3(jnp.pad(h3g[:, :h2s, :h2s, :], ((0, 0), (1, 1), (1, 1), (0, 0))),
                _w_conv(enc_w4), enc_b4, relu=True, bh=28)

    zflat = zg[:, :h2s, :h2s, :].reshape(bsz * h2s * h2s, _VQD)
    idx, zq, loss, perp = _vq(zflat, codebook)
    idxs = idx.reshape(bsz, h2s, h2s)

    q = h2s + 1
    bh1, wp1 = 8, 64
    nh1 = 8
    zq4 = zq.reshape(bsz, h2s, h2s, _VQD)
    zqp = jnp.pad(zq4, ((0, 0), (1, nh1 * bh1 + 1 - h2s - 1), (1, wp1 - h2s - 1),
                        (0, 0))).astype(bf16)
    zqp_flat = zqp.reshape(bsz, (nh1 * bh1 + 1) * wp1, _VQD)
    wf = dec_w1[:, :, ::-1, ::-1]
    t6 = jnp.transpose(wf, (2, 3, 1, 0)).reshape(2, 3, 2, 3, _VQD, 128)
    wd1 = jnp.transpose(t6, (0, 2, 4, 1, 3, 5)).reshape(4, _VQD, 9 * 128)
    bd1 = jnp.tile(dec_b1, 9)
    y1g = _dec1(zqp_flat, wd1.astype(bf16), bd1, bsz, wp1, bh1, nh1)
    y1g = y1g.reshape(bsz, nh1 * bh1, wp1, 3, 3, 128)[:, :q, :q]
    y1 = jnp.transpose(y1g, (0, 1, 3, 2, 4, 5)).reshape(bsz, 3 * q, 3 * q, 128)

    y2g = _conv3(jnp.pad(y1, ((0, 0), (2, 2), (2, 2), (0, 0))),
                 _w_conv(dec_w2, bf16), dec_b2, relu=True, bh=32, out_dtype=bf16)
    s2 = 3 * q + 2
    y3g = _conv3(jnp.pad(y2g[:, :s2, :s2, :], ((0, 0), (2, 2), (2, 2), (0, 0))),
                 _w_conv(dec_w3, bf16), dec_b3, relu=True, bh=32, out_dtype=bf16)

    _, hg3, wp3, _ = y3g.shape
    s3 = s2 + 2
    w4 = jnp.zeros((128, 8), jnp.float32).at[:, 0].set(dec_w4[0, :, 0, 0])
    b4 = jnp.zeros((8,), jnp.float32).at[0].set(dec_b4[0])
    y4 = _mm(y3g.reshape(bsz * hg3 * wp3, 128), w4.astype(bf16), b4, relu=False)
    decoded = y4[:, 0].reshape(bsz, hg3, wp3)[:, None, :s3, :s3]

    return loss.reshape(()), decoded, perp.reshape(()), idxs


# speedup vs baseline: 3.2864x; 1.2958x over previous
import functools

import jax
import jax.numpy as jnp
from jax.experimental import pallas as pl
from jax.experimental.pallas import tpu as pltpu

_BETA = 0.25
_VQK = 512
_VQD = 64


def _mm_kernel(x_ref, w_ref, b_ref, o_ref, *, relu, out_dtype):
    acc = jnp.dot(x_ref[...], w_ref[...], preferred_element_type=jnp.float32)
    acc = acc + b_ref[...]
    if relu:
        acc = jnp.maximum(acc, 0.0)
    o_ref[...] = acc.astype(out_dtype)


def _mm(x, w, b, relu, bm=512, out_dtype=jnp.float32):
    m, k = x.shape
    n = w.shape[1]
    return pl.pallas_call(
        functools.partial(_mm_kernel, relu=relu, out_dtype=out_dtype),
        grid=(pl.cdiv(m, bm),),
        in_specs=[
            pl.BlockSpec((bm, k), lambda i: (i, 0)),
            pl.BlockSpec((k, n), lambda i: (0, 0)),
            pl.BlockSpec((1, n), lambda i: (0, 0)),
        ],
        out_specs=pl.BlockSpec((bm, n), lambda i: (i, 0)),
        out_shape=jax.ShapeDtypeStruct((m, n), out_dtype),
    )(x, w, b.reshape(1, n))


def _enc2_kernel(x0_ref, x1_ref, x2_ref, s_ref, w_ref, b_ref, o_ref):
    acc = None
    for a, r in enumerate((x0_ref, x1_ref, x2_ref)):
        u = r[0, 0]
        for c in range(3):
            t = jnp.dot(s_ref[c], u, preferred_element_type=jnp.float32)
            t = jnp.dot(t, w_ref[3 * a + c], preferred_element_type=jnp.float32)
            acc = t if acc is None else acc + t
    o_ref[0, 0] = jnp.maximum(acc + b_ref[...], 0.0)


def _conv_s2(h1p, w9, b, ho):
    bsz, _, wp_in, cin = h1p.shape
    cout = w9.shape[2]
    col = jax.lax.broadcasted_iota(jnp.int32, (3, ho, wp_in), 2)
    tgt = 2 * jax.lax.broadcasted_iota(jnp.int32, (3, ho, wp_in), 1) + \
        jax.lax.broadcasted_iota(jnp.int32, (3, ho, wp_in), 0)
    sel = (col == tgt).astype(jnp.float32)
    return pl.pallas_call(
        _enc2_kernel,
        grid=(bsz, ho),
        in_specs=[
            pl.BlockSpec((1, 1, wp_in, cin), lambda i, r: (i, 2 * r, 0, 0)),
            pl.BlockSpec((1, 1, wp_in, cin), lambda i, r: (i, 2 * r + 1, 0, 0)),
            pl.BlockSpec((1, 1, wp_in, cin), lambda i, r: (i, 2 * r + 2, 0, 0)),
            pl.BlockSpec((3, ho, wp_in), lambda i, r: (0, 0, 0)),
            pl.BlockSpec((9, cin, cout), lambda i, r: (0, 0, 0)),
            pl.BlockSpec((1, cout), lambda i, r: (0, 0)),
        ],
        out_specs=pl.BlockSpec((1, 1, ho, cout), lambda i, r: (i, r, 0, 0)),
        out_shape=jax.ShapeDtypeStruct((bsz, ho, ho, cout), jnp.float32),
    )(h1p, h1p, h1p, sel, w9, b.reshape(1, cout))


def _conv3f_kernel(x_ref, w_ref, b_ref, *rest, wp, bh, m, slab, nh, nt,
                   relu, out_dtype, proj):
    if proj:
        w4_ref, b4_ref, o_ref, o2_ref, scr, sem = rest
    else:
        o_ref, scr, sem = rest
        w4_ref = b4_ref = o2_ref = None
    bi = pl.program_id(0)
    j = pl.program_id(1)
    t = bi * nh + j
    slot = jax.lax.rem(t, 2)

    def _start(tt, sl):
        bi2 = jax.lax.div(tt, nh)
        j2 = jax.lax.rem(tt, nh)
        pltpu.make_async_copy(x_ref.at[bi2, pl.ds(j2 * bh * wp, slab)],
                              scr.at[sl], sem.at[sl]).start()

    @pl.when(t == 0)
    def _first():
        _start(t, slot)

    @pl.when(t + 1 < nt)
    def _prefetch():
        _start(t + 1, jax.lax.rem(t + 1, 2))

    pltpu.make_async_copy(x_ref.at[bi, pl.ds(j * bh * wp, slab)],
                          scr.at[slot], sem.at[slot]).wait()
    acc = None
    for a in range(3):
        for c in range(3):
            off = a * wp + c
            xm = scr[slot, off:off + m, :]
            u = jnp.dot(xm, w_ref[3 * a + c], preferred_element_type=jnp.float32)
            acc = u if acc is None else acc + u
    acc = acc + b_ref[...]
    if relu:
        acc = jnp.maximum(acc, 0.0)
    o_ref[0, 0, :m, :] = acc.astype(out_dtype)
    if proj:
        y4 = jnp.dot(acc.astype(out_dtype), w4_ref[...],
                     preferred_element_type=jnp.float32)
        o2_ref[0, 0, :m, :] = y4 + b4_ref[...]


def _conv3(xpad, w9, b, relu, bh, out_dtype=jnp.float32, proj=None):
    bsz, hp, wp0, cin = xpad.shape
    ho = hp - 2
    cout = w9.shape[2]
    wp = -(-wp0 // 8) * 8
    nh = -(-ho // bh)
    hg = nh * bh
    xp2 = jnp.pad(xpad, ((0, 0), (0, hg + 2 - hp), (0, wp - wp0), (0, 0)))
    flat = xp2.reshape(bsz, (hg + 2) * wp, cin)
    m = bh * wp - 2
    slab = (bh + 2) * wp
    in_specs = [
        pl.BlockSpec(memory_space=pl.ANY),
        pl.BlockSpec((9, cin, cout), lambda i, j: (0, 0, 0)),
        pl.BlockSpec((1, cout), lambda i, j: (0, 0)),
    ]
    out_specs = [pl.BlockSpec((1, 1, bh * wp, cout), lambda i, j: (i, j, 0, 0))]
    out_shape = [jax.ShapeDtypeStruct((bsz, nh, bh * wp, cout), out_dtype)]
    args = [flat, w9, b.reshape(1, cout)]
    if proj is not None:
        w4, b4 = proj
        n2 = w4.shape[1]
        in_specs += [pl.BlockSpec((cout, n2), lambda i, j: (0, 0)),
                     pl.BlockSpec((1, n2), lambda i, j: (0, 0))]
        out_specs += [pl.BlockSpec((1, 1, bh * wp, n2), lambda i, j: (i, j, 0, 0))]
        out_shape += [jax.ShapeDtypeStruct((bsz, nh, bh * wp, n2), jnp.float32)]
        args += [w4, b4.reshape(1, n2)]
    out = pl.pallas_call(
        functools.partial(_conv3f_kernel, wp=wp, bh=bh, m=m, slab=slab,
                          nh=nh, nt=bsz * nh, relu=relu, out_dtype=out_dtype,
                          proj=proj is not None),
        grid=(bsz, nh),
        in_specs=in_specs,
        out_specs=out_specs,
        out_shape=out_shape,
        scratch_shapes=[pltpu.VMEM((2, slab, cin), xpad.dtype),
                        pltpu.SemaphoreType.DMA((2,))],
    )(*args)
    if proj is None:
        return out[0].reshape(bsz, hg, wp, cout)
    return (out[0].reshape(bsz, hg, wp, cout),
            out[1].reshape(bsz, hg, wp, n2))


def _dec1_kernel(x_ref, w_ref, b_ref, o_ref, scr, sem, *, wp, bh, m, slab):
    bi = pl.program_id(0)
    j = pl.program_id(1)
    cp = pltpu.make_async_copy(x_ref.at[bi, pl.ds(j * bh * wp, slab)], scr, sem)
    cp.start()
    cp.wait()
    offs = (wp + 1, wp, 1, 0)
    acc = None
    for t in range(4):
        xm = scr[offs[t]:offs[t] + m, :]
        u = jnp.dot(xm, w_ref[t], preferred_element_type=jnp.float32)
        acc = u if acc is None else acc + u
    acc = jnp.maximum(acc + b_ref[...], 0.0)
    o_ref[0, 0, :m, :] = acc.astype(jnp.bfloat16)


def _dec1(zqp_flat, wd1, bd1, bsz, wp, bh, nh):
    m = bh * wp - 1
    slab = (bh + 1) * wp
    return pl.pallas_call(
        functools.partial(_dec1_kernel, wp=wp, bh=bh, m=m, slab=slab),
        grid=(bsz, nh),
        in_specs=[
            pl.BlockSpec(memory_space=pl.ANY),
            pl.BlockSpec((4, _VQD, 1152), lambda i, j: (0, 0, 0)),
            pl.BlockSpec((1, 1152), lambda i, j: (0, 0)),
        ],
        out_specs=pl.BlockSpec((1, 1, bh * wp, 1152), lambda i, j: (i, j, 0, 0)),
        out_shape=jax.ShapeDtypeStruct((bsz, nh, bh * wp, 1152), jnp.bfloat16),
        scratch_shapes=[pltpu.VMEM((slab, _VQD), jnp.bfloat16),
                        pltpu.SemaphoreType.DMA],
    )(zqp_flat, wd1, bd1.reshape(1, 1152))


def _vq_kernel(z_ref, cbt_ref, cb_ref, idx_ref, zq_ref, cnt_ref, loss_ref,
               perp_ref, *, nblocks, m_total):
    i = pl.program_id(0)
    z = z_ref[...]
    cbt = cbt_ref[...]
    zn = jnp.sum(z * z, axis=1, keepdims=True)
    cn = jnp.sum(cbt * cbt, axis=0, keepdims=True)
    mm = jnp.dot(z, cbt, preferred_element_type=jnp.float32)
    d = zn + cn - 2.0 * mm
    dmin = jnp.min(d, axis=1, keepdims=True)
    col = jax.lax.broadcasted_iota(jnp.int32, d.shape, 1)
    idx = jnp.min(jnp.where(d == dmin, col, _VQK), axis=1, keepdims=True)
    idx_ref[...] = idx
    onehot = (col == idx).astype(jnp.float32)
    zq = jnp.dot(onehot, cb_ref[...], preferred_element_type=jnp.float32)
    zq_ref[...] = zq
    diff = zq - z
    sq = jnp.sum(jnp.sum(diff * diff, axis=1, keepdims=True),
                 axis=0, keepdims=True)
    cnt = jnp.sum(onehot, axis=0, keepdims=True)

    @pl.when(i == 0)
    def _init():
        cnt_ref[...] = cnt
        loss_ref[...] = sq

    @pl.when(i > 0)
    def _accum():
        cnt_ref[...] = cnt_ref[...] + cnt
        loss_ref[...] = loss_ref[...] + sq

    @pl.when(i == nblocks - 1)
    def _finalize():
        loss_ref[...] = (1.0 + _BETA) * loss_ref[...] / (m_total * _VQD)
        e = cnt_ref[...] / m_total
        ent = jnp.sum(e * jnp.log(e + 1e-10), axis=1, keepdims=True)
        perp_ref[...] = jnp.exp(-ent)


def _vq(zflat, codebook, bm=896):
    m = zflat.shape[0]
    nblocks = m // bm
    assert nblocks * bm == m
    kern = functools.partial(_vq_kernel, nblocks=nblocks, m_total=m)
    idx, zq, _cnt, loss, perp = pl.pallas_call(
        kern,
        grid=(nblocks,),
        in_specs=[
            pl.BlockSpec((bm, _VQD), lambda i: (i, 0)),
            pl.BlockSpec((_VQD, _VQK), lambda i: (0, 0)),
            pl.BlockSpec((_VQK, _VQD), lambda i: (0, 0)),
        ],
        out_specs=[
            pl.BlockSpec((bm, 1), lambda i: (i, 0)),
            pl.BlockSpec((bm, _VQD), lambda i: (i, 0)),
            pl.BlockSpec((1, _VQK), lambda i: (0, 0)),
            pl.BlockSpec((1, 1), lambda i: (0, 0)),
            pl.BlockSpec((1, 1), lambda i: (0, 0)),
        ],
        out_shape=[
            jax.ShapeDtypeStruct((m, 1), jnp.int32),
            jax.ShapeDtypeStruct((m, _VQD), jnp.float32),
            jax.ShapeDtypeStruct((1, _VQK), jnp.float32),
            jax.ShapeDtypeStruct((1, 1), jnp.float32),
            jax.ShapeDtypeStruct((1, 1), jnp.float32),
        ],
    )(zflat, codebook.T, codebook)
    return idx, zq, loss, perp


def _w_conv(w, dtype=jnp.float32):
    o, i, _, _ = w.shape
    return jnp.transpose(w, (2, 3, 1, 0)).reshape(9, i, o).astype(dtype)


def _w_im2col(w):
    o = w.shape[0]
    return jnp.transpose(w, (2, 3, 1, 0)).reshape(-1, o)


def kernel(x, enc_w1, enc_b1, enc_w2, enc_b2, enc_w3, enc_b3, enc_w4, enc_b4,
           codebook, dec_w1, dec_b1, dec_w2, dec_b2, dec_w3, dec_b3, dec_w4,
           dec_b4):
    bsz = x.shape[0]
    img = x.shape[2]
    h1s = img // 2
    h2s = h1s // 2
    bf16 = jnp.bfloat16

    xp = jnp.pad(x[:, 0], ((0, 0), (1, 1), (1, 1)))
    pat = jnp.stack(
        [xp[:, a:a + 2 * h1s:2, c:c + 2 * h1s:2] for a in range(3) for c in range(3)],
        axis=-1).reshape(bsz * h1s * h1s, 9)
    h1 = _mm(pat, _w_im2col(enc_w1), enc_b1, relu=True, bm=2048)
    h1 = h1.reshape(bsz, h1s, h1s, 128)

    h1p = jnp.pad(h1, ((0, 0), (1, 1), (1, 7), (0, 0)))
    h2 = _conv_s2(h1p, _w_conv(enc_w2), enc_b2, h2s)

    h3g = _conv3(jnp.pad(h2, ((0, 0), (1, 1), (1, 1), (0, 0))),
                 _w_conv(enc_w3), enc_b3, relu=True, bh=28)
    zg = _conv3(jnp.pad(h3g[:, :h2s, :h2s, :], ((0, 0), (1, 1), (1, 1), (0, 0))),
                _w_conv(enc_w4), enc_b4, relu=True, bh=28)

    zflat = zg[:, :h2s, :h2s, :].reshape(bsz * h2s * h2s, _VQD)
    idx, zq, loss, perp = _vq(zflat, codebook)
    idxs = idx.reshape(bsz, h2s, h2s)

    q = h2s + 1
    bh1, wp1 = 16, 64
    nh1 = 4
    zq4 = zq.reshape(bsz, h2s, h2s, _VQD)
    zqp = jnp.pad(zq4, ((0, 0), (1, nh1 * bh1 + 1 - h2s - 1), (1, wp1 - h2s - 1),
                        (0, 0))).astype(bf16)
    zqp_flat = zqp.reshape(bsz, (nh1 * bh1 + 1) * wp1, _VQD)
    wf = dec_w1[:, :, ::-1, ::-1]
    t6 = jnp.transpose(wf, (2, 3, 1, 0)).reshape(2, 3, 2, 3, _VQD, 128)
    wd1 = jnp.transpose(t6, (0, 2, 4, 1, 3, 5)).reshape(4, _VQD, 9 * 128)
    bd1 = jnp.tile(dec_b1, 9)
    y1g = _dec1(zqp_flat, wd1.astype(bf16), bd1, bsz, wp1, bh1, nh1)
    y1g = y1g.reshape(bsz, nh1 * bh1, wp1, 3, 3, 128)[:, :q, :q]
    y1 = jnp.transpose(y1g, (0, 1, 3, 2, 4, 5)).reshape(bsz, 3 * q, 3 * q, 128)

    y2g = _conv3(jnp.pad(y1, ((0, 0), (2, 2), (2, 2), (0, 0))),
                 _w_conv(dec_w2, bf16), dec_b2, relu=True, bh=32, out_dtype=bf16)
    s2 = 3 * q + 2
    s3 = s2 + 2
    w4 = jnp.zeros((128, 8), jnp.float32).at[:, 0].set(dec_w4[0, :, 0, 0])
    b4 = jnp.zeros((8,), jnp.float32).at[0].set(dec_b4[0])
    _, y4g = _conv3(jnp.pad(y2g[:, :s2, :s2, :], ((0, 0), (2, 2), (2, 2), (0, 0))),
                    _w_conv(dec_w3, bf16), dec_b3, relu=True, bh=32,
                    out_dtype=bf16, proj=(w4.astype(bf16), b4))
    decoded = y4g[..., 0][:, None, :s3, :s3]

    return loss.reshape(()), decoded, perp.reshape(()), idxs


# speedup vs baseline: 3.7298x; 1.1349x over previous
import functools

import jax
import jax.numpy as jnp
from jax.experimental import pallas as pl
from jax.experimental.pallas import tpu as pltpu

_BETA = 0.25
_VQK = 512
_VQD = 64


def _mm_kernel(x_ref, w_ref, b_ref, o_ref, *, relu, out_dtype):
    acc = jnp.dot(x_ref[...], w_ref[...], preferred_element_type=jnp.float32)
    acc = acc + b_ref[...]
    if relu:
        acc = jnp.maximum(acc, 0.0)
    o_ref[...] = acc.astype(out_dtype)


def _mm(x, w, b, relu, bm=512, out_dtype=jnp.float32):
    m, k = x.shape
    n = w.shape[1]
    return pl.pallas_call(
        functools.partial(_mm_kernel, relu=relu, out_dtype=out_dtype),
        grid=(pl.cdiv(m, bm),),
        in_specs=[
            pl.BlockSpec((bm, k), lambda i: (i, 0)),
            pl.BlockSpec((k, n), lambda i: (0, 0)),
            pl.BlockSpec((1, n), lambda i: (0, 0)),
        ],
        out_specs=pl.BlockSpec((bm, n), lambda i: (i, 0)),
        out_shape=jax.ShapeDtypeStruct((m, n), out_dtype),
    )(x, w, b.reshape(1, n))


def _enc2f_kernel(x_ref, w_ref, b_ref, o_ref, scr, sem, *, qq, wq, m, bsz):
    bi = pl.program_id(0)
    plane = qq * wq
    for p in range(4):
        pltpu.make_async_copy(x_ref.at[pl.ds((p * bsz + bi) * plane, plane)],
                              scr.at[p], sem.at[p]).start()
    for p in range(4):
        pltpu.make_async_copy(x_ref.at[pl.ds((p * bsz + bi) * plane, plane)],
                              scr.at[p], sem.at[p]).wait()
    acc = None
    for a in range(3):
        for c in range(3):
            p = (a % 2) * 2 + (c % 2)
            off = (a // 2) * wq + (c // 2)
            xm = scr[p, off:off + m, :]
            u = jnp.dot(xm, w_ref[3 * a + c], preferred_element_type=jnp.float32)
            acc = u if acc is None else acc + u
    o_ref[0, :m, :] = jnp.maximum(acc + b_ref[...], 0.0)


def _conv_s2(h1phases, w9, b, qq, bsz):
    cin = h1phases.shape[1]
    cout = w9.shape[2]
    m = qq * qq - qq - 1
    mo8 = -(-m // 8) * 8
    return pl.pallas_call(
        functools.partial(_enc2f_kernel, qq=qq, wq=qq, m=m, bsz=bsz),
        grid=(bsz,),
        in_specs=[
            pl.BlockSpec(memory_space=pl.ANY),
            pl.BlockSpec((9, cin, cout), lambda i: (0, 0, 0)),
            pl.BlockSpec((1, cout), lambda i: (0, 0)),
        ],
        out_specs=pl.BlockSpec((1, mo8, cout), lambda i: (i, 0, 0)),
        out_shape=jax.ShapeDtypeStruct((bsz, mo8, cout), jnp.float32),
        scratch_shapes=[pltpu.VMEM((4, qq * qq, cin), jnp.float32),
                        pltpu.SemaphoreType.DMA((4,))],
    )(h1phases, w9, b.reshape(1, cout))


def _conv3f_kernel(x_ref, w_ref, b_ref, *rest, wp, bh, m, slab, nh, nt,
                   relu, out_dtype, proj):
    if proj:
        w4_ref, b4_ref, o_ref, o2_ref, scr, sem = rest
    else:
        o_ref, scr, sem = rest
        w4_ref = b4_ref = o2_ref = None
    bi = pl.program_id(0)
    j = pl.program_id(1)
    t = bi * nh + j
    slot = jax.lax.rem(t, 2)

    def _start(tt, sl):
        bi2 = jax.lax.div(tt, nh)
        j2 = jax.lax.rem(tt, nh)
        pltpu.make_async_copy(x_ref.at[bi2, pl.ds(j2 * bh * wp, slab)],
                              scr.at[sl], sem.at[sl]).start()

    @pl.when(t == 0)
    def _first():
        _start(t, slot)

    @pl.when(t + 1 < nt)
    def _prefetch():
        _start(t + 1, jax.lax.rem(t + 1, 2))

    pltpu.make_async_copy(x_ref.at[bi, pl.ds(j * bh * wp, slab)],
                          scr.at[slot], sem.at[slot]).wait()
    acc = None
    for a in range(3):
        for c in range(3):
            off = a * wp + c
            xm = scr[slot, off:off + m, :]
            u = jnp.dot(xm, w_ref[3 * a + c], preferred_element_type=jnp.float32)
            acc = u if acc is None else acc + u
    acc = acc + b_ref[...]
    if relu:
        acc = jnp.maximum(acc, 0.0)
    o_ref[0, 0, :m, :] = acc.astype(out_dtype)
    if proj:
        y4 = jnp.dot(acc.astype(out_dtype), w4_ref[...],
                     preferred_element_type=jnp.float32)
        o2_ref[0, 0, :m, :] = y4 + b4_ref[...]


def _conv3(xpad, w9, b, relu, bh, out_dtype=jnp.float32, proj=None):
    bsz, hp, wp0, cin = xpad.shape
    ho = hp - 2
    cout = w9.shape[2]
    wp = -(-wp0 // 8) * 8
    nh = -(-ho // bh)
    hg = nh * bh
    xp2 = jnp.pad(xpad, ((0, 0), (0, hg + 2 - hp), (0, wp - wp0), (0, 0)))
    flat = xp2.reshape(bsz, (hg + 2) * wp, cin)
    m = bh * wp - 2
    slab = (bh + 2) * wp
    in_specs = [
        pl.BlockSpec(memory_space=pl.ANY),
        pl.BlockSpec((9, cin, cout), lambda i, j: (0, 0, 0)),
        pl.BlockSpec((1, cout), lambda i, j: (0, 0)),
    ]
    out_specs = [pl.BlockSpec((1, 1, bh * wp, cout), lambda i, j: (i, j, 0, 0))]
    out_shape = [jax.ShapeDtypeStruct((bsz, nh, bh * wp, cout), out_dtype)]
    args = [flat, w9, b.reshape(1, cout)]
    if proj is not None:
        w4, b4 = proj
        n2 = w4.shape[1]
        in_specs += [pl.BlockSpec((cout, n2), lambda i, j: (0, 0)),
                     pl.BlockSpec((1, n2), lambda i, j: (0, 0))]
        out_specs += [pl.BlockSpec((1, 1, bh * wp, n2), lambda i, j: (i, j, 0, 0))]
        out_shape += [jax.ShapeDtypeStruct((bsz, nh, bh * wp, n2), jnp.float32)]
        args += [w4, b4.reshape(1, n2)]
    out = pl.pallas_call(
        functools.partial(_conv3f_kernel, wp=wp, bh=bh, m=m, slab=slab,
                          nh=nh, nt=bsz * nh, relu=relu, out_dtype=out_dtype,
                          proj=proj is not None),
        grid=(bsz, nh),
        in_specs=in_specs,
        out_specs=out_specs,
        out_shape=out_shape,
        scratch_shapes=[pltpu.VMEM((2, slab, cin), xpad.dtype),
                        pltpu.SemaphoreType.DMA((2,))],
    )(*args)
    if proj is None:
        return out[0].reshape(bsz, hg, wp, cout)
    return (out[0].reshape(bsz, hg, wp, cout),
            out[1].reshape(bsz, hg, wp, n2))


def _dec1_kernel(x_ref, w_ref, b_ref, o_ref, scr, sem, *, wp, bh, m, slab):
    bi = pl.program_id(0)
    j = pl.program_id(1)
    cp = pltpu.make_async_copy(x_ref.at[bi, pl.ds(j * bh * wp, slab)], scr, sem)
    cp.start()
    cp.wait()
    offs = (wp + 1, wp, 1, 0)
    acc = None
    for t in range(4):
        xm = scr[offs[t]:offs[t] + m, :]
        u = jnp.dot(xm, w_ref[t], preferred_element_type=jnp.float32)
        acc = u if acc is None else acc + u
    acc = jnp.maximum(acc + b_ref[...], 0.0)
    o_ref[0, 0, :m, :] = acc.astype(jnp.bfloat16)


def _dec1(zqp_flat, wd1, bd1, bsz, wp, bh, nh):
    m = bh * wp - 1
    slab = (bh + 1) * wp
    return pl.pallas_call(
        functools.partial(_dec1_kernel, wp=wp, bh=bh, m=m, slab=slab),
        grid=(bsz, nh),
        in_specs=[
            pl.BlockSpec(memory_space=pl.ANY),
            pl.BlockSpec((4, _VQD, 1152), lambda i, j: (0, 0, 0)),
            pl.BlockSpec((1, 1152), lambda i, j: (0, 0)),
        ],
        out_specs=pl.BlockSpec((1, 1, bh * wp, 1152), lambda i, j: (i, j, 0, 0)),
        out_shape=jax.ShapeDtypeStruct((bsz, nh, bh * wp, 1152), jnp.bfloat16),
        scratch_shapes=[pltpu.VMEM((slab, _VQD), jnp.bfloat16),
                        pltpu.SemaphoreType.DMA],
    )(zqp_flat, wd1, bd1.reshape(1, 1152))


def _vq_kernel(z_ref, cbt_ref, cb_ref, idx_ref, zq_ref, cnt_ref, loss_ref,
               perp_ref, *, nblocks, m_total):
    i = pl.program_id(0)
    z = z_ref[...]
    cbt = cbt_ref[...]
    zn = jnp.sum(z * z, axis=1, keepdims=True)
    cn = jnp.sum(cbt * cbt, axis=0, keepdims=True)
    mm = jnp.dot(z, cbt, preferred_element_type=jnp.float32)
    d = zn + cn - 2.0 * mm
    dmin = jnp.min(d, axis=1, keepdims=True)
    col = jax.lax.broadcasted_iota(jnp.int32, d.shape, 1)
    idx = jnp.min(jnp.where(d == dmin, col, _VQK), axis=1, keepdims=True)
    idx_ref[...] = idx
    onehot = (col == idx).astype(jnp.float32)
    zq = jnp.dot(onehot, cb_ref[...], preferred_element_type=jnp.float32)
    zq_ref[...] = zq
    diff = zq - z
    sq = jnp.sum(jnp.sum(diff * diff, axis=1, keepdims=True),
                 axis=0, keepdims=True)
    cnt = jnp.sum(onehot, axis=0, keepdims=True)

    @pl.when(i == 0)
    def _init():
        cnt_ref[...] = cnt
        loss_ref[...] = sq

    @pl.when(i > 0)
    def _accum():
        cnt_ref[...] = cnt_ref[...] + cnt
        loss_ref[...] = loss_ref[...] + sq

    @pl.when(i == nblocks - 1)
    def _finalize():
        loss_ref[...] = (1.0 + _BETA) * loss_ref[...] / (m_total * _VQD)
        e = cnt_ref[...] / m_total
        ent = jnp.sum(e * jnp.log(e + 1e-10), axis=1, keepdims=True)
        perp_ref[...] = jnp.exp(-ent)


def _vq(zflat, codebook, bm=896):
    m = zflat.shape[0]
    nblocks = m // bm
    assert nblocks * bm == m
    kern = functools.partial(_vq_kernel, nblocks=nblocks, m_total=m)
    idx, zq, _cnt, loss, perp = pl.pallas_call(
        kern,
        grid=(nblocks,),
        in_specs=[
            pl.BlockSpec((bm, _VQD), lambda i: (i, 0)),
            pl.BlockSpec((_VQD, _VQK), lambda i: (0, 0)),
            pl.BlockSpec((_VQK, _VQD), lambda i: (0, 0)),
        ],
        out_specs=[
            pl.BlockSpec((bm, 1), lambda i: (i, 0)),
            pl.BlockSpec((bm, _VQD), lambda i: (i, 0)),
            pl.BlockSpec((1, _VQK), lambda i: (0, 0)),
            pl.BlockSpec((1, 1), lambda i: (0, 0)),
            pl.BlockSpec((1, 1), lambda i: (0, 0)),
        ],
        out_shape=[
            jax.ShapeDtypeStruct((m, 1), jnp.int32),
            jax.ShapeDtypeStruct((m, _VQD), jnp.float32),
            jax.ShapeDtypeStruct((1, _VQK), jnp.float32),
            jax.ShapeDtypeStruct((1, 1), jnp.float32),
            jax.ShapeDtypeStruct((1, 1), jnp.float32),
        ],
    )(zflat, codebook.T, codebook)
    return idx, zq, loss, perp


def _w_conv(w, dtype=jnp.float32):
    o, i, _, _ = w.shape
    return jnp.transpose(w, (2, 3, 1, 0)).reshape(9, i, o).astype(dtype)


def _w_im2col(w):
    o = w.shape[0]
    return jnp.transpose(w, (2, 3, 1, 0)).reshape(-1, o)


def kernel(x, enc_w1, enc_b1, enc_w2, enc_b2, enc_w3, enc_b3, enc_w4, enc_b4,
           codebook, dec_w1, dec_b1, dec_w2, dec_b2, dec_w3, dec_b3, dec_w4,
           dec_b4):
    bsz = x.shape[0]
    img = x.shape[2]
    h1s = img // 2
    h2s = h1s // 2
    bf16 = jnp.bfloat16

    qq = h2s + 1
    xp3 = jnp.pad(x[:, 0], ((0, 0), (3, 3), (3, 3)))
    r57 = jax.lax.broadcasted_iota(jnp.int32, (qq, qq), 0)
    c57 = jax.lax.broadcasted_iota(jnp.int32, (qq, qq), 1)
    phases = []
    for dy in range(2):
        for dx in range(2):
            mask2 = (((2 * r57 + dy >= 1) & (2 * r57 + dy <= 2 * h1s)) &
                     ((2 * c57 + dx >= 1) & (2 * c57 + dx <= 2 * h1s)))
            maskb = jnp.broadcast_to(mask2.astype(jnp.float32)[None],
                                     (bsz, qq, qq))
            cols = []
            for a in range(3):
                for c in range(3):
                    sy, sx = 2 * dy + a, 2 * dx + c
                    cols.append(xp3[:, sy:sy + 225:4, sx:sx + 225:4] * maskb)
            cols.append(maskb)
            phases.append(jnp.stack(cols, axis=-1))
    pat = jnp.stack(phases, 0).reshape(4 * bsz * qq * qq, 10)
    w10 = jnp.concatenate([_w_im2col(enc_w1), enc_b1.reshape(1, 128)], 0)
    h1mm = _mm(pat, w10, jnp.zeros((128,), jnp.float32), relu=True, bm=2048)

    h2f = _conv_s2(h1mm, _w_conv(enc_w2), enc_b2, qq, bsz)
    h2 = jnp.pad(h2f, ((0, 0), (0, qq * qq - h2f.shape[1]), (0, 0)))
    h2 = h2.reshape(bsz, qq, qq, 128)[:, :h2s, :h2s, :]

    h3g = _conv3(jnp.pad(h2, ((0, 0), (1, 1), (1, 1), (0, 0))),
                 _w_conv(enc_w3), enc_b3, relu=True, bh=28)
    zg = _conv3(jnp.pad(h3g[:, :h2s, :h2s, :], ((0, 0), (1, 1), (1, 1), (0, 0))),
                _w_conv(enc_w4), enc_b4, relu=True, bh=28)

    zflat = zg[:, :h2s, :h2s, :].reshape(bsz * h2s * h2s, _VQD)
    idx, zq, loss, perp = _vq(zflat, codebook)
    idxs = idx.reshape(bsz, h2s, h2s)

    q = h2s + 1
    bh1, wp1 = 16, 64
    nh1 = 4
    zq4 = zq.reshape(bsz, h2s, h2s, _VQD)
    zqp = jnp.pad(zq4, ((0, 0), (1, nh1 * bh1 + 1 - h2s - 1), (1, wp1 - h2s - 1),
                        (0, 0))).astype(bf16)
    zqp_flat = zqp.reshape(bsz, (nh1 * bh1 + 1) * wp1, _VQD)
    wf = dec_w1[:, :, ::-1, ::-1]
    t6 = jnp.transpose(wf, (2, 3, 1, 0)).reshape(2, 3, 2, 3, _VQD, 128)
    wd1 = jnp.transpose(t6, (0, 2, 4, 1, 3, 5)).reshape(4, _VQD, 9 * 128)
    bd1 = jnp.tile(dec_b1, 9)
    y1g = _dec1(zqp_flat, wd1.astype(bf16), bd1, bsz, wp1, bh1, nh1)
    y1g = y1g.reshape(bsz, nh1 * bh1, wp1, 3, 3, 128)[:, :q, :q]
    y1 = jnp.transpose(y1g, (0, 1, 3, 2, 4, 5)).reshape(bsz, 3 * q, 3 * q, 128)

    y2g = _conv3(jnp.pad(y1, ((0, 0), (2, 2), (2, 2), (0, 0))),
                 _w_conv(dec_w2, bf16), dec_b2, relu=True, bh=32, out_dtype=bf16)
    s2 = 3 * q + 2
    s3 = s2 + 2
    w4 = jnp.zeros((128, 8), jnp.float32).at[:, 0].set(dec_w4[0, :, 0, 0])
    b4 = jnp.zeros((8,), jnp.float32).at[0].set(dec_b4[0])
    _, y4g = _conv3(jnp.pad(y2g[:, :s2, :s2, :], ((0, 0), (2, 2), (2, 2), (0, 0))),
                    _w_conv(dec_w3, bf16), dec_b3, relu=True, bh=32,
                    out_dtype=bf16, proj=(w4.astype(bf16), b4))
    decoded = y4g[..., 0][:, None, :s3, :s3]

    return loss.reshape(()), decoded, perp.reshape(()), idxs
